# baseline (speedup 1.0000x reference)
import functools
import math

import jax
import jax.numpy as jnp
from jax import lax
from jax.experimental import pallas as pl
from jax.experimental.pallas import tpu as pltpu
from jax.experimental.pallas import tpu_sc as plsc

D_MODEL = 768
EMB_DIM = 64
SCALE = math.sqrt(D_MODEL)

_NC = 2
_NS = 16
_NW = _NC * _NS

_CHUNK = 128


def _sc_gather(idx, table, n_rows):
    rows_per_w = n_rows // _NW
    n_chunks = rows_per_w // _CHUNK

    mesh = plsc.VectorSubcoreMesh(core_axis_name="c", subcore_axis_name="s")

    @functools.partial(
        pl.kernel,
        mesh=mesh,
        out_type=jax.ShapeDtypeStruct((n_rows, EMB_DIM), jnp.float32),
        compiler_params=pltpu.CompilerParams(use_tc_tiling_on_sc=False),
        scratch_types=[
            pltpu.VMEM((rows_per_w,), jnp.int32),
            pltpu.VMEM((rows_per_w, EMB_DIM), jnp.float32),
            pltpu.SemaphoreType.DMA,
        ],
    )
    def gather_kernel(idx_hbm, table_hbm, out_hbm, idx_v, rows_v, sem):
        wid = lax.axis_index("s") * _NC + lax.axis_index("c")
        base = wid * rows_per_w
        pltpu.sync_copy(idx_hbm.at[pl.ds(base, rows_per_w)], idx_v)
        handles = []
        for j in range(n_chunks):
            sl = pl.ds(j * _CHUNK, _CHUNK)
            handles.append(
                pltpu.async_copy(table_hbm.at[idx_v.at[sl]], rows_v.at[sl], sem)
            )
        for h in handles:
            h.wait()
        pltpu.sync_copy(rows_v, out_hbm.at[pl.ds(base, rows_per_w)])

    return gather_kernel(idx, table)


def _tc_project(emb, wt, b2, n_rows, block_rows):
    def body(emb_ref, wt_ref, b_ref, out_ref):
        acc = jnp.dot(emb_ref[...], wt_ref[...],
                      preferred_element_type=jnp.float32)
        out_ref[...] = (acc + b_ref[...]) * SCALE

    return pl.pallas_call(
        body,
        grid=(n_rows // block_rows,),
        in_specs=[
            pl.BlockSpec((block_rows, EMB_DIM), lambda i: (i, 0)),
            pl.BlockSpec((EMB_DIM, D_MODEL), lambda i: (0, 0)),
            pl.BlockSpec((1, D_MODEL), lambda i: (0, 0)),
        ],
        out_specs=pl.BlockSpec((block_rows, D_MODEL), lambda i: (i, 0)),
        out_shape=jax.ShapeDtypeStruct((n_rows, D_MODEL), jnp.float32),
    )(emb, wt, b2)


def kernel(x, table, W, b):
    B, L = x.shape
    n_rows = B * L
    idx = x.reshape(n_rows).astype(jnp.int32)
    emb = _sc_gather(idx, table, n_rows)
    wt = W.T
    b2 = b.reshape(1, D_MODEL)
    out = _tc_project(emb, wt, b2, n_rows, block_rows=1024)
    return out.reshape(B, L, D_MODEL)
